# baseline (speedup 1.0000x reference)
import jax
import jax.numpy as jnp
from jax import lax
from jax.experimental import pallas as pl
from jax.experimental.pallas import tpu as pltpu
from jax.experimental.pallas import tpu_sc as plsc

B = 4096
N = 256
KEEP = 128
LANES = 16
NUM_WORKERS = 32
ROWS_PER_WORKER = B // NUM_WORKERS
GROUPS = ROWS_PER_WORKER // LANES
WORDS = N // 32
TC_BLK = 8

_ROTS = ((13, 15, 26, 6), (17, 29, 16, 24))


def _threefry_bits(x1):
    ks0 = jnp.uint32(0)
    ks1 = jnp.uint32(1)
    ks2 = jnp.uint32(0x1BD11BDA) ^ ks0 ^ ks1
    ks = (ks0, ks1, ks2)
    x0 = jnp.zeros_like(x1) + ks0
    x1 = x1 + ks1
    for i in range(5):
        for r in _ROTS[i % 2]:
            x0 = x0 + x1
            x1 = (x1 << jnp.uint32(r)) | (x1 >> jnp.uint32(32 - r))
            x1 = x0 ^ x1
        x0 = x0 + ks[(i + 1) % 3]
        x1 = x1 + ks[(i + 2) % 3] + jnp.uint32(i + 1)
    return x0 ^ x1


def _tc_keys_kernel(out_ref):
    blk = pl.program_id(0)
    shape = (TC_BLK, N, ROWS_PER_WORKER)
    w3 = lax.broadcasted_iota(jnp.uint32, shape, 0)
    j3 = lax.broadcasted_iota(jnp.uint32, shape, 1)
    r3 = lax.broadcasted_iota(jnp.uint32, shape, 2)
    wid = w3 + jnp.uint32(TC_BLK) * blk.astype(jnp.uint32)
    flat = (wid * jnp.uint32(ROWS_PER_WORKER) + r3) * jnp.uint32(N) + j3
    out_ref[...] = _threefry_bits(flat) >> jnp.uint32(9)


def _tc_keys():
    return pl.pallas_call(
        _tc_keys_kernel,
        out_shape=jax.ShapeDtypeStruct((NUM_WORKERS, N, ROWS_PER_WORKER), jnp.uint32),
        grid=(NUM_WORKERS // TC_BLK,),
        out_specs=pl.BlockSpec((TC_BLK, N, ROWS_PER_WORKER), lambda b: (b, 0, 0)),
    )()


def _select_rows(vals_v, packed_v):
    one = jnp.ones((LANES,), jnp.uint32)
    zero = jnp.zeros((LANES,), jnp.uint32)
    topbit = zero + jnp.uint32(0x80000000)

    def group_body(g, carry):
        r0 = g * LANES

        def bit_body(b, vc):
            v, c_lt = vc
            shift = (jnp.int32(22) - b).astype(jnp.uint32)
            bit = lax.shift_left(jnp.uint32(1), shift)
            trial = v + bit

            def cnt_body(jb, cnt):
                j0 = jb * 16
                for k in range(16):
                    kv = vals_v[j0 + k, pl.ds(r0, LANES)]
                    cnt = cnt + jnp.where(kv < trial, one, zero)
                return cnt

            cnt = lax.fori_loop(0, N // 16, cnt_body, zero)
            acc = cnt < jnp.uint32(KEEP)
            v = v + jnp.where(acc, bit, jnp.uint32(0))
            c_lt = jnp.where(acc, cnt, c_lt)
            return (v, c_lt)

        v, c_lt = lax.fori_loop(0, 23, bit_body, (zero, zero))
        quota = jnp.uint32(KEEP) - c_lt

        def emit_word(w, t):
            def word_body(jj, wt):
                word, t = wt
                for k in range(8):
                    kv = vals_v[w * 32 + jj * 8 + k, pl.ds(r0, LANES)]
                    is_eq = kv == v
                    keep = (kv < v) | (is_eq & (t < quota))
                    t = t + jnp.where(is_eq, one, zero)
                    word = (word >> jnp.uint32(1)) | jnp.where(
                        keep, topbit, zero
                    )
                return (word, t)

            word, t = lax.fori_loop(0, 4, word_body, (zero, t))
            packed_v[w, pl.ds(r0, LANES)] = word
            return t

        t = zero
        for w in range(WORDS):
            t = emit_word(w, t)
        return carry

    lax.fori_loop(0, GROUPS, group_body, jnp.int32(0))


def _sc_select_kernel(vals_hbm, out_hbm, vals_v, packed_v):
    wid = lax.axis_index("s") * 2 + lax.axis_index("c")
    pltpu.sync_copy(vals_hbm.at[wid], vals_v)
    _select_rows(vals_v, packed_v)
    pltpu.sync_copy(packed_v, out_hbm.at[wid])


@jax.jit
def _build_mask():
    vals = _tc_keys()
    mesh = plsc.VectorSubcoreMesh(core_axis_name="c", subcore_axis_name="s")
    k = pl.kernel(
        _sc_select_kernel,
        jax.ShapeDtypeStruct((NUM_WORKERS, WORDS, ROWS_PER_WORKER), jnp.uint32),
        mesh=mesh,
        scratch_types=[
            pltpu.VMEM((N, ROWS_PER_WORKER), jnp.uint32),
            pltpu.VMEM((WORDS, ROWS_PER_WORKER), jnp.uint32),
        ],
    )
    return k(vals)


def kernel(input):
    del input
    packed = _build_mask()
    per_row = packed.transpose(0, 2, 1).reshape(B, WORDS)
    shifts = jnp.arange(32, dtype=jnp.uint32)[None, None, :]
    bits = (per_row[:, :, None] >> shifts) & jnp.uint32(1)
    return bits.reshape(B, N).astype(jnp.bool_).reshape(B, N, 1)


# speedup vs baseline: 1.0122x; 1.0122x over previous
import jax
import jax.numpy as jnp
from jax import lax
from jax.experimental import pallas as pl
from jax.experimental.pallas import tpu as pltpu
from jax.experimental.pallas import tpu_sc as plsc

B = 4096
N = 256
KEEP = 128
LANES = 16
NUM_WORKERS = 32
ROWS_PER_WORKER = B // NUM_WORKERS
GROUPS = ROWS_PER_WORKER // LANES
WORDS = N // 32
TC_BLK = 8

_ROTS = ((13, 15, 26, 6), (17, 29, 16, 24))


def _threefry_bits(x1):
    ks0 = jnp.uint32(0)
    ks1 = jnp.uint32(1)
    ks2 = jnp.uint32(0x1BD11BDA) ^ ks0 ^ ks1
    ks = (ks0, ks1, ks2)
    x0 = jnp.zeros_like(x1) + ks0
    x1 = x1 + ks1
    for i in range(5):
        for r in _ROTS[i % 2]:
            x0 = x0 + x1
            x1 = (x1 << jnp.uint32(r)) | (x1 >> jnp.uint32(32 - r))
            x1 = x0 ^ x1
        x0 = x0 + ks[(i + 1) % 3]
        x1 = x1 + ks[(i + 2) % 3] + jnp.uint32(i + 1)
    return x0 ^ x1


def _tc_keys_kernel(out_ref):
    blk = pl.program_id(0)
    shape = (TC_BLK, N, ROWS_PER_WORKER)
    w3 = lax.broadcasted_iota(jnp.uint32, shape, 0)
    j3 = lax.broadcasted_iota(jnp.uint32, shape, 1)
    r3 = lax.broadcasted_iota(jnp.uint32, shape, 2)
    wid = w3 + jnp.uint32(TC_BLK) * blk.astype(jnp.uint32)
    flat = (wid * jnp.uint32(ROWS_PER_WORKER) + r3) * jnp.uint32(N) + j3
    out_ref[...] = _threefry_bits(flat) >> jnp.uint32(9)


def _tc_keys():
    return pl.pallas_call(
        _tc_keys_kernel,
        out_shape=jax.ShapeDtypeStruct((NUM_WORKERS, N, ROWS_PER_WORKER), jnp.uint32),
        grid=(NUM_WORKERS // TC_BLK,),
        out_specs=pl.BlockSpec((TC_BLK, N, ROWS_PER_WORKER), lambda b: (b, 0, 0)),
    )()


def _select_rows(vals_v, packed_v):
    one = jnp.ones((LANES,), jnp.uint32)
    zero = jnp.zeros((LANES,), jnp.uint32)
    topbit = zero + jnp.uint32(0x80000000)

    def group_body(g, carry):
        r0 = g * LANES

        def bit_body(b, vc):
            v, c_lt = vc
            shift = (jnp.int32(22) - b).astype(jnp.uint32)
            bit = lax.shift_left(jnp.uint32(1), shift)
            trial = v + bit

            def cnt_body(jb, cnt):
                j0 = jb * 16
                for k in range(16):
                    kv = vals_v[j0 + k, pl.ds(r0, LANES)]
                    cnt = cnt + jnp.where(kv < trial, one, zero)
                return cnt

            cnt = lax.fori_loop(0, N // 16, cnt_body, zero)
            acc = cnt < jnp.uint32(KEEP)
            v = v + jnp.where(acc, bit, jnp.uint32(0))
            c_lt = jnp.where(acc, cnt, c_lt)
            return (v, c_lt)

        v, c_lt = lax.fori_loop(0, 23, bit_body, (zero, zero))
        quota = jnp.uint32(KEEP) - c_lt

        def emit_word(w, t):
            def word_body(jj, wt):
                word, t = wt
                for k in range(4):
                    kv = vals_v[w * 32 + jj * 4 + k, pl.ds(r0, LANES)]
                    is_eq = kv == v
                    keep = (kv < v) | (is_eq & (t < quota))
                    t = t + jnp.where(is_eq, one, zero)
                    word = (word >> jnp.uint32(1)) | jnp.where(
                        keep, topbit, zero
                    )
                return (word, t)

            word, t = lax.fori_loop(0, 8, word_body, (zero, t))
            packed_v[w, pl.ds(r0, LANES)] = word
            return t

        t = zero
        for w in range(WORDS):
            t = emit_word(w, t)
        return carry

    lax.fori_loop(0, GROUPS, group_body, jnp.int32(0))


def _sc_select_kernel(vals_hbm, out_hbm, vals_v, packed_v):
    wid = lax.axis_index("s") * 2 + lax.axis_index("c")
    pltpu.sync_copy(vals_hbm.at[wid], vals_v)
    _select_rows(vals_v, packed_v)
    pltpu.sync_copy(packed_v, out_hbm.at[wid])


@jax.jit
def _build_mask():
    vals = _tc_keys()
    mesh = plsc.VectorSubcoreMesh(core_axis_name="c", subcore_axis_name="s")
    k = pl.kernel(
        _sc_select_kernel,
        jax.ShapeDtypeStruct((NUM_WORKERS, WORDS, ROWS_PER_WORKER), jnp.uint32),
        mesh=mesh,
        scratch_types=[
            pltpu.VMEM((N, ROWS_PER_WORKER), jnp.uint32),
            pltpu.VMEM((WORDS, ROWS_PER_WORKER), jnp.uint32),
        ],
    )
    return k(vals)


def kernel(input):
    del input
    packed = _build_mask()
    per_row = packed.transpose(0, 2, 1).reshape(B, WORDS)
    shifts = jnp.arange(32, dtype=jnp.uint32)[None, None, :]
    bits = (per_row[:, :, None] >> shifts) & jnp.uint32(1)
    return bits.reshape(B, N).astype(jnp.bool_).reshape(B, N, 1)
